# baseline (speedup 1.0000x reference)
import jax
import jax.numpy as jnp
from jax.experimental import pallas as pl
from jax.experimental.pallas import tpu as pltpu

E = 8
M0 = M1 = 64
N0 = N1 = 64
D_IN = M0 * M1
D_OUT = N0 * N1

BT = 128


def _w1_prep_kernel(w1_ref, out_ref):
    out_ref[...] = jnp.swapaxes(w1_ref[...], 1, 2).astype(jnp.bfloat16)


def _fused_kernel(x_ref, gw_ref, w1_ref, w2_ref, b_ref, out_ref):
    x = x_ref[...]

    gw = gw_ref[...]
    dn = (((1,), (1,)), ((), ()))
    g = jax.lax.dot_general(x, gw, dn, preferred_element_type=jnp.float32)

    neg = jnp.float32(-3.4e38)
    lane = jax.lax.broadcasted_iota(jnp.int32, (BT, E), 1)
    m1 = jnp.max(g, axis=1, keepdims=True)
    eq1 = g == m1
    idx1 = jnp.min(jnp.where(eq1, lane, E), axis=1, keepdims=True)
    first1 = lane == idx1
    g2 = jnp.where(first1, neg, g)
    m2 = jnp.max(g2, axis=1, keepdims=True)
    eq2 = g2 == m2
    idx2 = jnp.min(jnp.where(eq2, lane, E), axis=1, keepdims=True)
    first2 = lane == idx2
    e2 = jnp.exp(m2 - m1)
    denom = 1.0 + e2
    w_top1 = 1.0 / denom + 1e-6
    w_top2 = e2 / denom + 1e-6
    sw = jnp.where(first1, w_top1, jnp.where(first2, w_top2, 0.0))

    xt = x.astype(jnp.bfloat16).T.reshape(M1, M0, BT)

    t = jax.lax.dot_general(
        w1_ref[...], xt,
        dimension_numbers=(((2,), (1,)), ((0,), (0,))),
        preferred_element_type=jnp.float32,
    )

    swt = jnp.concatenate([sw.T] * N0, axis=0).astype(jnp.bfloat16)
    tg = t.astype(jnp.bfloat16) * swt[None, :, :]

    w2 = w2_ref[...]
    cols = []
    for k in range(N0):
        lhsk = tg[:, k * E:(k + 1) * E, :].reshape(M1 * E, BT)
        yk = jax.lax.dot_general(
            lhsk, w2[k],
            dimension_numbers=(((0,), (0,)), ((), ())),
            preferred_element_type=jnp.float32,
        )
        cols.append(yk)
    out = jnp.concatenate(cols, axis=1) + b_ref[...]
    out_ref[...] = out


@jax.jit
def kernel(inputs, gate_W, W1, W2, b):
    batch_shape = inputs.shape[:-1]
    x = inputs.reshape(-1, D_IN)
    T = x.shape[0]
    w1r = pl.pallas_call(
        _w1_prep_kernel,
        out_shape=jax.ShapeDtypeStruct((M1, N0 * E, M0), jnp.bfloat16),
    )(W1.reshape(M1, M0, N0 * E))
    w2r = W2.reshape(N0, M1 * E, N1)
    b2 = b.reshape(1, D_OUT)

    grid = (T // BT,)
    out = pl.pallas_call(
        _fused_kernel,
        grid=grid,
        in_specs=[
            pl.BlockSpec((BT, D_IN), lambda i: (i, 0)),
            pl.BlockSpec((E, D_IN), lambda i: (0, 0)),
            pl.BlockSpec((M1, N0 * E, M0), lambda i: (0, 0, 0)),
            pl.BlockSpec((N0, M1 * E, N1), lambda i: (0, 0, 0)),
            pl.BlockSpec((1, D_OUT), lambda i: (0, 0)),
        ],
        out_specs=pl.BlockSpec((BT, D_OUT), lambda i: (i, 0)),
        out_shape=jax.ShapeDtypeStruct((T, D_OUT), jnp.float32),
    )(x, gate_W, w1r, w2r, b2)
    return out.reshape(*batch_shape, D_OUT)


# speedup vs baseline: 1.2816x; 1.2816x over previous
import jax
import jax.numpy as jnp
from jax.experimental import pallas as pl
from jax.experimental.pallas import tpu as pltpu

E = 8
M0 = M1 = 64
N0 = N1 = 64
D_IN = M0 * M1
D_OUT = N0 * N1

BT = 128


def _fused_kernel(x_ref, gw_ref, w1r_ref, w2_ref, b_ref, out_ref):
    x = x_ref[...]

    gw = gw_ref[...]
    dn = (((1,), (1,)), ((), ()))
    g = jax.lax.dot_general(x, gw, dn, preferred_element_type=jnp.float32)

    neg = jnp.float32(-3.4e38)
    lane = jax.lax.broadcasted_iota(jnp.int32, (BT, E), 1)
    m1 = jnp.max(g, axis=1, keepdims=True)
    eq1 = g == m1
    idx1 = jnp.min(jnp.where(eq1, lane, E), axis=1, keepdims=True)
    first1 = lane == idx1
    g2 = jnp.where(first1, neg, g)
    m2 = jnp.max(g2, axis=1, keepdims=True)
    eq2 = g2 == m2
    idx2 = jnp.min(jnp.where(eq2, lane, E), axis=1, keepdims=True)
    first2 = lane == idx2
    e2 = jnp.exp(m2 - m1)
    denom = 1.0 + e2
    w_top1 = 1.0 / denom + 1e-6
    w_top2 = e2 / denom + 1e-6
    sw = jnp.where(first1, w_top1, jnp.where(first2, w_top2, 0.0))

    xt = x.astype(jnp.bfloat16).T.reshape(M1, M0, BT)

    t = jax.lax.dot_general(
        w1r_ref[...], xt,
        dimension_numbers=(((2,), (1,)), ((0,), (0,))),
        preferred_element_type=jnp.float32,
    )

    swt = jnp.concatenate([sw.T] * N0, axis=0).astype(jnp.bfloat16)
    tg = t.astype(jnp.bfloat16) * swt[None, :, :]

    w2 = w2_ref[...]
    rows = []
    for k in range(N0):
        rhsk = tg[:, k * E:(k + 1) * E, :].reshape(M1 * E, BT)
        yk = jax.lax.dot_general(
            w2[k], rhsk,
            dimension_numbers=(((1,), (0,)), ((), ())),
            preferred_element_type=jnp.float32,
        )
        rows.append(yk)
    yt = jnp.concatenate(rows, axis=0)
    out = yt.T + b_ref[...]
    out_ref[...] = out


@jax.jit
def kernel(inputs, gate_W, W1, W2, b):
    batch_shape = inputs.shape[:-1]
    x = inputs.reshape(-1, D_IN)
    T = x.shape[0]
    w1r = W1.transpose(0, 2, 3, 1).reshape(M1, N0 * E, M0).astype(jnp.bfloat16)
    w2r = W2.transpose(0, 3, 1, 2).reshape(N0, N1, M1 * E).astype(jnp.bfloat16)
    b2 = b.reshape(1, D_OUT)

    grid = (T // BT,)
    out = pl.pallas_call(
        _fused_kernel,
        grid=grid,
        in_specs=[
            pl.BlockSpec((BT, D_IN), lambda i: (i, 0)),
            pl.BlockSpec((E, D_IN), lambda i: (0, 0)),
            pl.BlockSpec((M1, N0 * E, M0), lambda i: (0, 0, 0)),
            pl.BlockSpec((N0, N1, M1 * E), lambda i: (0, 0, 0)),
            pl.BlockSpec((1, D_OUT), lambda i: (0, 0)),
        ],
        out_specs=pl.BlockSpec((BT, D_OUT), lambda i: (i, 0)),
        out_shape=jax.ShapeDtypeStruct((T, D_OUT), jnp.float32),
    )(x, gate_W, w1r, w2r, b2)
    return out.reshape(*batch_shape, D_OUT)
